# Initial kernel scaffold; baseline (speedup 1.0000x reference)
import functools

import jax
import jax.numpy as jnp
from jax import lax
from jax.experimental import pallas as pl
from jax.experimental.pallas import tpu as pltpu
from jax.experimental.pallas import tpu_sc as plsc

N = 10000
E = 320000
D = 128
DOUT = 16
EPS = 1e-5

NCORES = 2
NSUB = 16
NWORK = NCORES * NSUB
CHUNK = 128
NCHUNK = -(-E // (NWORK * CHUNK))
EPT = NCHUNK * CHUNK
EPAD = EPT * NWORK
NPAD = 10240
STRIPE = NPAD // NSUB

BS = 1000
GRID = N // BS

_sc_mesh = plsc.VectorSubcoreMesh(core_axis_name="c", subcore_axis_name="s")


def _make_deg_kernel():
    @functools.partial(
        pl.kernel,
        out_type=jax.ShapeDtypeStruct((NCORES, NPAD), jnp.float32),
        mesh=_sc_mesh,
        scratch_types=[
            pltpu.VMEM((CHUNK,), jnp.int32),
            pltpu.VMEM((CHUNK,), jnp.float32),
            pltpu.VMEM_SHARED((NPAD,), jnp.float32),
        ],
    )
    def deg_kernel(dstp_hbm, out_hbm, dst_v, ones_v, acc_sh):
        c = lax.axis_index("c")
        s = lax.axis_index("s")

        @pl.loop(0, CHUNK // 16)
        def _(j):
            ones_v[pl.ds(j * 16, 16)] = jnp.zeros((16,), jnp.float32)

        for k in range(STRIPE // CHUNK):
            pltpu.sync_copy(ones_v, acc_sh.at[pl.ds(s * STRIPE + k * CHUNK, CHUNK)])

        @pl.loop(0, CHUNK // 16)
        def _(j):
            ones_v[pl.ds(j * 16, 16)] = jnp.ones((16,), jnp.float32)

        plsc.subcore_barrier()

        wbase = (c * NSUB + s) * EPT

        @pl.loop(0, NCHUNK)
        def _(i):
            base = wbase + i * CHUNK
            pltpu.sync_copy(dstp_hbm.at[pl.ds(base, CHUNK)], dst_v)
            pltpu.sync_copy(ones_v, acc_sh.at[dst_v], add=True)

        plsc.subcore_barrier()
        pltpu.sync_copy(acc_sh.at[pl.ds(s * STRIPE, STRIPE)],
                        out_hbm.at[c, pl.ds(s * STRIPE, STRIPE)])

    return deg_kernel


def _make_agg_kernel(d):
    @functools.partial(
        pl.kernel,
        out_type=jax.ShapeDtypeStruct((NCORES, NPAD, d), jnp.float32),
        mesh=_sc_mesh,
        scratch_types=[
            pltpu.VMEM((CHUNK,), jnp.int32),
            pltpu.VMEM((CHUNK,), jnp.int32),
            pltpu.VMEM((CHUNK, d), jnp.float32),
            pltpu.VMEM_SHARED((NPAD, d), jnp.float32),
        ],
    )
    def agg_kernel(hs_hbm, srcp_hbm, dstp_hbm, out_hbm, src_v, dst_v, rows_v, acc_sh):
        c = lax.axis_index("c")
        s = lax.axis_index("s")

        @pl.loop(0, CHUNK)
        def _(i):
            for j in range(d // 16):
                rows_v[i, pl.ds(j * 16, 16)] = jnp.zeros((16,), jnp.float32)

        for k in range(STRIPE // CHUNK):
            pltpu.sync_copy(rows_v, acc_sh.at[pl.ds(s * STRIPE + k * CHUNK, CHUNK)])

        plsc.subcore_barrier()

        wbase = (c * NSUB + s) * EPT

        @pl.loop(0, NCHUNK)
        def _(i):
            base = wbase + i * CHUNK
            pltpu.sync_copy(srcp_hbm.at[pl.ds(base, CHUNK)], src_v)
            pltpu.sync_copy(dstp_hbm.at[pl.ds(base, CHUNK)], dst_v)
            pltpu.sync_copy(hs_hbm.at[src_v], rows_v)
            pltpu.sync_copy(rows_v, acc_sh.at[dst_v], add=True)

        plsc.subcore_barrier()
        pltpu.sync_copy(acc_sh.at[pl.ds(s * STRIPE, STRIPE)],
                        out_hbm.at[c, pl.ds(s * STRIPE, STRIPE)])

    return agg_kernel


_deg_call = _make_deg_kernel()
_agg128 = _make_agg_kernel(D)
_agg16 = _make_agg_kernel(DOUT)


def _tc1_body(x_ref, w_ref, degT_ref, hs_ref, dinv_ref):
    t = jnp.dot(x_ref[...], w_ref[...], preferred_element_type=jnp.float32)
    deg = degT_ref[:, 0:1] + degT_ref[:, 1:2] + 1.0
    dinv = lax.rsqrt(deg)
    dinvb = jnp.broadcast_to(dinv, t.shape)
    hs_ref[...] = t * dinvb
    dinv_ref[...] = dinvb


def _tc1(x, W1, degT):
    return pl.pallas_call(
        _tc1_body,
        grid=(GRID,),
        in_specs=[
            pl.BlockSpec((BS, D), lambda i: (i, 0)),
            pl.BlockSpec((D, D), lambda i: (0, 0)),
            pl.BlockSpec((BS, 2), lambda i: (i, 0)),
        ],
        out_specs=[
            pl.BlockSpec((BS, D), lambda i: (i, 0)),
            pl.BlockSpec((BS, D), lambda i: (i, 0)),
        ],
        out_shape=[
            jax.ShapeDtypeStruct((N, D), jnp.float32),
            jax.ShapeDtypeStruct((N, D), jnp.float32),
        ],
    )(x, W1, degT)


def _tcmid_body(parts_ref, hs_ref, dinv_ref, b_ref, g_ref, bb_ref, w_ref, out_ref):
    p = parts_ref[...]
    dv = dinv_ref[...]
    y = dv * (p[0] + p[1] + hs_ref[...]) + b_ref[...]
    mean = jnp.mean(y, axis=1, keepdims=True)
    yc = y - mean
    var = jnp.mean(yc * yc, axis=1, keepdims=True)
    yn = yc * lax.rsqrt(var + EPS) * g_ref[...] + bb_ref[...]
    r = jnp.maximum(yn, 0.0)
    t = jnp.dot(r, w_ref[...], preferred_element_type=jnp.float32)
    out_ref[...] = t * dv[:, : t.shape[1]]


def _tcmid(parts, hs, dinv, b, g, bb, W, dout):
    return pl.pallas_call(
        _tcmid_body,
        grid=(GRID,),
        in_specs=[
            pl.BlockSpec((NCORES, BS, D), lambda i: (0, i, 0)),
            pl.BlockSpec((BS, D), lambda i: (i, 0)),
            pl.BlockSpec((BS, D), lambda i: (i, 0)),
            pl.BlockSpec((1, D), lambda i: (0, 0)),
            pl.BlockSpec((1, D), lambda i: (0, 0)),
            pl.BlockSpec((1, D), lambda i: (0, 0)),
            pl.BlockSpec((D, dout), lambda i: (0, 0)),
        ],
        out_specs=pl.BlockSpec((BS, dout), lambda i: (i, 0)),
        out_shape=jax.ShapeDtypeStruct((N, dout), jnp.float32),
    )(parts, hs, dinv, b, g, bb, W)


def _tclast_body(parts_ref, hs_ref, dinv_ref, b_ref, out_ref):
    p = parts_ref[...]
    dv = dinv_ref[...][:, :DOUT]
    y = dv * (p[0] + p[1] + hs_ref[...]) + b_ref[...]
    m = jnp.max(y, axis=1, keepdims=True)
    ym = y - m
    out_ref[...] = ym - jnp.log(jnp.sum(jnp.exp(ym), axis=1, keepdims=True))


def _tclast(parts, hs, dinv, b):
    return pl.pallas_call(
        _tclast_body,
        grid=(GRID,),
        in_specs=[
            pl.BlockSpec((NCORES, BS, DOUT), lambda i: (0, i, 0)),
            pl.BlockSpec((BS, DOUT), lambda i: (i, 0)),
            pl.BlockSpec((BS, D), lambda i: (i, 0)),
            pl.BlockSpec((1, DOUT), lambda i: (0, 0)),
        ],
        out_specs=pl.BlockSpec((BS, DOUT), lambda i: (i, 0)),
        out_shape=jax.ShapeDtypeStruct((N, DOUT), jnp.float32),
    )(parts, hs, dinv, b)


def kernel(x, edge_index, W1, b1, W2, b2, W3, b3, ln1_g, ln1_b, ln2_g, ln2_b):
    src = edge_index[0]
    dst = edge_index[1]

    npadrows = NPAD - N
    padi = jnp.arange(EPAD - E, dtype=jnp.int32)
    src_p = jnp.concatenate([src, padi % N])
    dst_p = jnp.concatenate([dst, N + padi % npadrows])

    parts_deg = _deg_call(dst_p)
    degT = parts_deg[:, :N].T

    hs1, dinv = _tc1(x, W1, degT)
    parts1 = _agg128(hs1, src_p, dst_p)
    hs2 = _tcmid(parts1, hs1, dinv, b1.reshape(1, D), ln1_g.reshape(1, D),
                 ln1_b.reshape(1, D), W2, D)
    parts2 = _agg128(hs2, src_p, dst_p)
    hs3 = _tcmid(parts2, hs2, dinv, b2.reshape(1, D), ln2_g.reshape(1, D),
                 ln2_b.reshape(1, D), W3, DOUT)
    parts3 = _agg16(hs3, src_p, dst_p)
    return _tclast(parts3, hs3, dinv, b3.reshape(1, DOUT))


# speedup vs baseline: 13.8896x; 13.8896x over previous
import functools

import jax
import jax.numpy as jnp
from jax import lax
from jax.experimental import pallas as pl
from jax.experimental.pallas import tpu as pltpu
from jax.experimental.pallas import tpu_sc as plsc

N = 10000
E = 320000
D = 128
DOUT = 16
EPS = 1e-5

NCORES = 2
NSUB = 16
NWORK = NCORES * NSUB
CHUNK = 128
NCHUNK = -(-E // (NWORK * CHUNK))
EPT = NCHUNK * CHUNK
EPAD = EPT * NWORK
NPAD = 10240
STRIPE = NPAD // NSUB

BS = 1000
GRID = N // BS

_sc_mesh = plsc.VectorSubcoreMesh(core_axis_name="c", subcore_axis_name="s")


def _make_deg_kernel():
    @functools.partial(
        pl.kernel,
        out_type=jax.ShapeDtypeStruct((NCORES, NPAD), jnp.float32),
        mesh=_sc_mesh,
        scratch_types=[
            pltpu.VMEM((CHUNK,), jnp.int32),
            pltpu.VMEM((CHUNK,), jnp.float32),
            pltpu.VMEM_SHARED((NPAD,), jnp.float32),
        ],
    )
    def deg_kernel(dstp_hbm, out_hbm, dst_v, ones_v, acc_sh):
        c = lax.axis_index("c")
        s = lax.axis_index("s")

        @pl.loop(0, CHUNK // 16)
        def _(j):
            ones_v[pl.ds(j * 16, 16)] = jnp.zeros((16,), jnp.float32)

        for k in range(STRIPE // CHUNK):
            pltpu.sync_copy(ones_v, acc_sh.at[pl.ds(s * STRIPE + k * CHUNK, CHUNK)])

        @pl.loop(0, CHUNK // 16)
        def _(j):
            ones_v[pl.ds(j * 16, 16)] = jnp.ones((16,), jnp.float32)

        plsc.subcore_barrier()

        wbase = (c * NSUB + s) * EPT

        @pl.loop(0, NCHUNK)
        def _(i):
            base = wbase + i * CHUNK
            pltpu.sync_copy(dstp_hbm.at[pl.ds(base, CHUNK)], dst_v)
            pltpu.sync_copy(ones_v, acc_sh.at[dst_v], add=True)

        plsc.subcore_barrier()
        pltpu.sync_copy(acc_sh.at[pl.ds(s * STRIPE, STRIPE)],
                        out_hbm.at[c, pl.ds(s * STRIPE, STRIPE)])

    return deg_kernel


def _make_agg_kernel(d):
    @functools.partial(
        pl.kernel,
        out_type=jax.ShapeDtypeStruct((NCORES, NPAD, d), jnp.float32),
        mesh=_sc_mesh,
        scratch_types=[
            pltpu.VMEM((CHUNK,), jnp.int32),
            pltpu.VMEM((CHUNK,), jnp.int32),
            pltpu.VMEM((CHUNK, d), jnp.float32),
            pltpu.VMEM_SHARED((NPAD, d), jnp.float32),
        ],
    )
    def agg_kernel(hs_hbm, srcp_hbm, dstp_hbm, out_hbm, src_v, dst_v, rows_v, acc_sh):
        c = lax.axis_index("c")
        s = lax.axis_index("s")

        @pl.loop(0, CHUNK)
        def _(i):
            for j in range(d // 16):
                rows_v[i, pl.ds(j * 16, 16)] = jnp.zeros((16,), jnp.float32)

        for k in range(STRIPE // CHUNK):
            pltpu.sync_copy(rows_v, acc_sh.at[pl.ds(s * STRIPE + k * CHUNK, CHUNK)])

        plsc.subcore_barrier()

        wbase = (c * NSUB + s) * EPT

        @pl.loop(0, NCHUNK)
        def _(i):
            base = wbase + i * CHUNK
            pltpu.sync_copy(srcp_hbm.at[pl.ds(base, CHUNK)], src_v)
            pltpu.sync_copy(dstp_hbm.at[pl.ds(base, CHUNK)], dst_v)
            pltpu.sync_copy(hs_hbm.at[src_v], rows_v)
            pltpu.sync_copy(rows_v, acc_sh.at[dst_v], add=True)

        plsc.subcore_barrier()
        pltpu.sync_copy(acc_sh.at[pl.ds(s * STRIPE, STRIPE)],
                        out_hbm.at[c, pl.ds(s * STRIPE, STRIPE)])

    return agg_kernel


_deg_call = _make_deg_kernel()
_agg128 = _make_agg_kernel(D)


def _tc1_body(x_ref, w_ref, degT_ref, hs_ref, dinv_ref):
    t = jnp.dot(x_ref[...], w_ref[...], preferred_element_type=jnp.float32)
    deg = degT_ref[:, 0:1] + degT_ref[:, 1:2] + 1.0
    dinv = lax.rsqrt(deg)
    dinvb = jnp.broadcast_to(dinv, t.shape)
    hs_ref[...] = t * dinvb
    dinv_ref[...] = dinvb


def _tc1(x, W1, degT):
    return pl.pallas_call(
        _tc1_body,
        grid=(GRID,),
        in_specs=[
            pl.BlockSpec((BS, D), lambda i: (i, 0)),
            pl.BlockSpec((D, D), lambda i: (0, 0)),
            pl.BlockSpec((BS, 2), lambda i: (i, 0)),
        ],
        out_specs=[
            pl.BlockSpec((BS, D), lambda i: (i, 0)),
            pl.BlockSpec((BS, D), lambda i: (i, 0)),
        ],
        out_shape=[
            jax.ShapeDtypeStruct((N, D), jnp.float32),
            jax.ShapeDtypeStruct((N, D), jnp.float32),
        ],
    )(x, W1, degT)


def _tcmid_body(parts_ref, hs_ref, dinv_ref, b_ref, g_ref, bb_ref, w_ref, out_ref):
    p = parts_ref[...]
    dv = dinv_ref[...]
    y = dv * (p[0] + p[1] + hs_ref[...]) + b_ref[...]
    mean = jnp.mean(y, axis=1, keepdims=True)
    yc = y - mean
    var = jnp.mean(yc * yc, axis=1, keepdims=True)
    yn = yc * lax.rsqrt(var + EPS) * g_ref[...] + bb_ref[...]
    r = jnp.maximum(yn, 0.0)
    t = jnp.dot(r, w_ref[...], preferred_element_type=jnp.float32)
    out_ref[...] = t * dv[:, : t.shape[1]]


def _tcmid(parts, hs, dinv, b, g, bb, W, dout):
    return pl.pallas_call(
        _tcmid_body,
        grid=(GRID,),
        in_specs=[
            pl.BlockSpec((NCORES, BS, D), lambda i: (0, i, 0)),
            pl.BlockSpec((BS, D), lambda i: (i, 0)),
            pl.BlockSpec((BS, D), lambda i: (i, 0)),
            pl.BlockSpec((1, D), lambda i: (0, 0)),
            pl.BlockSpec((1, D), lambda i: (0, 0)),
            pl.BlockSpec((1, D), lambda i: (0, 0)),
            pl.BlockSpec((D, dout), lambda i: (0, 0)),
        ],
        out_specs=pl.BlockSpec((BS, dout), lambda i: (i, 0)),
        out_shape=jax.ShapeDtypeStruct((N, dout), jnp.float32),
    )(parts, hs, dinv, b, g, bb, W)


def _tclast_body(parts_ref, hs_ref, dinv_ref, b_ref, out_ref):
    p = parts_ref[...]
    dv = dinv_ref[...][:, :DOUT]
    y = dv * (p[0, :, :DOUT] + p[1, :, :DOUT] + hs_ref[...][:, :DOUT]) + b_ref[...]
    m = jnp.max(y, axis=1, keepdims=True)
    ym = y - m
    out_ref[...] = ym - jnp.log(jnp.sum(jnp.exp(ym), axis=1, keepdims=True))


def _tclast(parts, hs, dinv, b):
    return pl.pallas_call(
        _tclast_body,
        grid=(GRID,),
        in_specs=[
            pl.BlockSpec((NCORES, BS, D), lambda i: (0, i, 0)),
            pl.BlockSpec((BS, D), lambda i: (i, 0)),
            pl.BlockSpec((BS, D), lambda i: (i, 0)),
            pl.BlockSpec((1, DOUT), lambda i: (0, 0)),
        ],
        out_specs=pl.BlockSpec((BS, DOUT), lambda i: (i, 0)),
        out_shape=jax.ShapeDtypeStruct((N, DOUT), jnp.float32),
    )(parts, hs, dinv, b)


def kernel(x, edge_index, W1, b1, W2, b2, W3, b3, ln1_g, ln1_b, ln2_g, ln2_b):
    src = edge_index[0]
    dst = edge_index[1]

    npadrows = NPAD - N
    padi = jnp.arange(EPAD - E, dtype=jnp.int32)
    src_p = jnp.concatenate([src, padi % N])
    dst_p = jnp.concatenate([dst, N + padi % npadrows])

    parts_deg = _deg_call(dst_p)
    degT = parts_deg[:, :N].T

    hs1, dinv = _tc1(x, W1, degT)
    parts1 = _agg128(hs1, src_p, dst_p)
    hs2 = _tcmid(parts1, hs1, dinv, b1.reshape(1, D), ln1_g.reshape(1, D),
                 ln1_b.reshape(1, D), W2, D)
    parts2 = _agg128(hs2, src_p, dst_p)
    W3p = jnp.pad(W3, ((0, 0), (0, D - DOUT)))
    hs3 = _tcmid(parts2, hs2, dinv, b2.reshape(1, D), ln2_g.reshape(1, D),
                 ln2_b.reshape(1, D), W3p, D)
    parts3 = _agg128(hs3, src_p, dst_p)
    return _tclast(parts3, hs3, dinv, b3.reshape(1, DOUT))


# speedup vs baseline: 28.7953x; 2.0732x over previous
import functools

import jax
import jax.numpy as jnp
from jax import lax
from jax.experimental import pallas as pl
from jax.experimental.pallas import tpu as pltpu
from jax.experimental.pallas import tpu_sc as plsc

N = 10000
E = 320000
D = 128
DOUT = 16
EPS = 1e-5

NCORES = 2
NSUB = 16
NWORK = NCORES * NSUB
CHUNK = 128
NCHUNK = 80
EPT = NCHUNK * CHUNK
EPAD = EPT * NWORK
NPAD = 10240
STRIPE = NPAD // NSUB
NBUF = 2
HALF = NCHUNK // 2

BS = 1000
GRID = N // BS

_sc_mesh = plsc.VectorSubcoreMesh(core_axis_name="c", subcore_axis_name="s")


def _make_deg_kernel():
    @functools.partial(
        pl.kernel,
        out_type=jax.ShapeDtypeStruct((NCORES, NPAD), jnp.float32),
        mesh=_sc_mesh,
        scratch_types=[
            pltpu.VMEM((NCHUNK, CHUNK), jnp.int32),
            pltpu.VMEM((CHUNK,), jnp.float32),
            pltpu.VMEM_SHARED((NPAD,), jnp.float32),
            pltpu.SemaphoreType.DMA,
            pltpu.SemaphoreType.DMA,
        ],
    )
    def deg_kernel(dstp_hbm, out_hbm, dstall_v, ones_v, acc_sh, isem, ssem):
        c = lax.axis_index("c")
        s = lax.axis_index("s")
        wid = c * NSUB + s

        icopy = pltpu.async_copy(dstp_hbm.at[pl.ds(wid * NCHUNK, NCHUNK)],
                                 dstall_v, isem)

        @pl.loop(0, CHUNK // 16)
        def _(j):
            ones_v[pl.ds(j * 16, 16)] = jnp.zeros((16,), jnp.float32)

        for k in range(STRIPE // CHUNK):
            pltpu.sync_copy(ones_v, acc_sh.at[pl.ds(s * STRIPE + k * CHUNK, CHUNK)])

        @pl.loop(0, CHUNK // 16)
        def _(j):
            ones_v[pl.ds(j * 16, 16)] = jnp.ones((16,), jnp.float32)

        icopy.wait()
        plsc.subcore_barrier()

        @pl.loop(0, NCHUNK, step=16)
        def _(i0):
            for b in range(16):
                pltpu.async_copy(ones_v, acc_sh.at[dstall_v.at[i0 + b]], ssem,
                                 add=True)
            for b in range(16):
                pltpu.make_async_copy(ones_v, acc_sh.at[dstall_v.at[i0]],
                                      ssem).wait()

        plsc.subcore_barrier()
        pltpu.sync_copy(acc_sh.at[pl.ds(s * STRIPE, STRIPE)],
                        out_hbm.at[c, pl.ds(s * STRIPE, STRIPE)])

    return deg_kernel


def _make_agg_kernel(d):
    @functools.partial(
        pl.kernel,
        out_type=jax.ShapeDtypeStruct((NCORES, NPAD, d), jnp.float32),
        mesh=_sc_mesh,
        scratch_types=(
            [
                pltpu.VMEM((HALF, CHUNK), jnp.int32),
                pltpu.VMEM((HALF, CHUNK), jnp.int32),
            ]
            + [pltpu.VMEM((CHUNK, d), jnp.float32) for _ in range(NBUF)]
            + [pltpu.VMEM_SHARED((NPAD, d), jnp.float32)]
            + [pltpu.SemaphoreType.DMA for _ in range(2 * NBUF + 1)]
        ),
    )
    def agg_kernel(hs_hbm, srcp_hbm, dstp_hbm, out_hbm, srcall_v, dstall_v,
                   r0, r1, acc_sh, g0, g1, s0, s1, isem):
        rows = (r0, r1)
        gsem = (g0, g1)
        ssem = (s0, s1)
        c = lax.axis_index("c")
        s = lax.axis_index("s")
        wid = c * NSUB + s

        ic1 = pltpu.async_copy(srcp_hbm.at[pl.ds(wid * NCHUNK, HALF)],
                               srcall_v, isem)
        ic2 = pltpu.async_copy(dstp_hbm.at[pl.ds(wid * NCHUNK, HALF)],
                               dstall_v, isem)

        @pl.loop(0, CHUNK)
        def _(i):
            for j in range(d // 16):
                r0[i, pl.ds(j * 16, 16)] = jnp.zeros((16,), jnp.float32)

        for k in range(STRIPE // CHUNK):
            pltpu.sync_copy(r0, acc_sh.at[pl.ds(s * STRIPE + k * CHUNK, CHUNK)])

        ic1.wait()
        ic2.wait()
        plsc.subcore_barrier()

        for h in range(NCHUNK // HALF):
            if h > 0:
                pltpu.sync_copy(
                    srcp_hbm.at[pl.ds(wid * NCHUNK + h * HALF, HALF)],
                    srcall_v)
                pltpu.sync_copy(
                    dstp_hbm.at[pl.ds(wid * NCHUNK + h * HALF, HALF)],
                    dstall_v)

            @pl.loop(0, HALF + NBUF, step=NBUF)
            def _(i0):
                for b in range(NBUF):
                    i = i0 + b

                    @pl.when(i < HALF)
                    def _():
                        @pl.when(i >= NBUF)
                        def _():
                            pltpu.make_async_copy(
                                rows[b], acc_sh.at[dstall_v.at[i - NBUF]],
                                ssem[b]).wait()
                        pltpu.async_copy(hs_hbm.at[srcall_v.at[i]], rows[b],
                                         gsem[b])

                    cc = i - 1
                    bb = (b - 1) % NBUF

                    @pl.when((cc >= 0) & (cc < HALF))
                    def _():
                        pltpu.make_async_copy(hs_hbm.at[srcall_v.at[cc]],
                                              rows[bb], gsem[bb]).wait()
                        pltpu.async_copy(rows[bb], acc_sh.at[dstall_v.at[cc]],
                                         ssem[bb], add=True)

            for b in range(NBUF):
                pltpu.make_async_copy(rows[b], acc_sh.at[dstall_v.at[0]],
                                      ssem[b]).wait()

        plsc.subcore_barrier()
        pltpu.sync_copy(acc_sh.at[pl.ds(s * STRIPE, STRIPE)],
                        out_hbm.at[c, pl.ds(s * STRIPE, STRIPE)])

    return agg_kernel


_deg_call = _make_deg_kernel()
_agg128 = _make_agg_kernel(D)


def _tc1_body(x_ref, w_ref, degT_ref, hs_ref, dinv_ref):
    t = jnp.dot(x_ref[...], w_ref[...], preferred_element_type=jnp.float32)
    deg = degT_ref[:, 0:1] + degT_ref[:, 1:2] + 1.0
    dinv = lax.rsqrt(deg)
    dinvb = jnp.broadcast_to(dinv, t.shape)
    hs_ref[...] = t * dinvb
    dinv_ref[...] = dinvb


def _tc1(x, W1, degT):
    return pl.pallas_call(
        _tc1_body,
        grid=(GRID,),
        in_specs=[
            pl.BlockSpec((BS, D), lambda i: (i, 0)),
            pl.BlockSpec((D, D), lambda i: (0, 0)),
            pl.BlockSpec((BS, 2), lambda i: (i, 0)),
        ],
        out_specs=[
            pl.BlockSpec((BS, D), lambda i: (i, 0)),
            pl.BlockSpec((BS, D), lambda i: (i, 0)),
        ],
        out_shape=[
            jax.ShapeDtypeStruct((N, D), jnp.float32),
            jax.ShapeDtypeStruct((N, D), jnp.float32),
        ],
    )(x, W1, degT)


def _tcmid_body(parts_ref, hs_ref, dinv_ref, b_ref, g_ref, bb_ref, w_ref, out_ref):
    p = parts_ref[...]
    dv = dinv_ref[...]
    y = dv * (p[0] + p[1] + hs_ref[...]) + b_ref[...]
    mean = jnp.mean(y, axis=1, keepdims=True)
    yc = y - mean
    var = jnp.mean(yc * yc, axis=1, keepdims=True)
    yn = yc * lax.rsqrt(var + EPS) * g_ref[...] + bb_ref[...]
    r = jnp.maximum(yn, 0.0)
    t = jnp.dot(r, w_ref[...], preferred_element_type=jnp.float32)
    out_ref[...] = t * dv[:, : t.shape[1]]


def _tcmid(parts, hs, dinv, b, g, bb, W, dout):
    return pl.pallas_call(
        _tcmid_body,
        grid=(GRID,),
        in_specs=[
            pl.BlockSpec((NCORES, BS, D), lambda i: (0, i, 0)),
            pl.BlockSpec((BS, D), lambda i: (i, 0)),
            pl.BlockSpec((BS, D), lambda i: (i, 0)),
            pl.BlockSpec((1, D), lambda i: (0, 0)),
            pl.BlockSpec((1, D), lambda i: (0, 0)),
            pl.BlockSpec((1, D), lambda i: (0, 0)),
            pl.BlockSpec((D, dout), lambda i: (0, 0)),
        ],
        out_specs=pl.BlockSpec((BS, dout), lambda i: (i, 0)),
        out_shape=jax.ShapeDtypeStruct((N, dout), jnp.float32),
    )(parts, hs, dinv, b, g, bb, W)


def _tclast_body(parts_ref, hs_ref, dinv_ref, b_ref, out_ref):
    p = parts_ref[...]
    dv = dinv_ref[...][:, :DOUT]
    y = dv * (p[0, :, :DOUT] + p[1, :, :DOUT] + hs_ref[...][:, :DOUT]) + b_ref[...]
    m = jnp.max(y, axis=1, keepdims=True)
    ym = y - m
    out_ref[...] = ym - jnp.log(jnp.sum(jnp.exp(ym), axis=1, keepdims=True))


def _tclast(parts, hs, dinv, b):
    return pl.pallas_call(
        _tclast_body,
        grid=(GRID,),
        in_specs=[
            pl.BlockSpec((NCORES, BS, D), lambda i: (0, i, 0)),
            pl.BlockSpec((BS, D), lambda i: (i, 0)),
            pl.BlockSpec((BS, D), lambda i: (i, 0)),
            pl.BlockSpec((1, DOUT), lambda i: (0, 0)),
        ],
        out_specs=pl.BlockSpec((BS, DOUT), lambda i: (i, 0)),
        out_shape=jax.ShapeDtypeStruct((N, DOUT), jnp.float32),
    )(parts, hs, dinv, b)


def kernel(x, edge_index, W1, b1, W2, b2, W3, b3, ln1_g, ln1_b, ln2_g, ln2_b):
    src = edge_index[0]
    dst = edge_index[1]

    npadrows = NPAD - N
    padi = jnp.arange(EPAD - E, dtype=jnp.int32)
    src_p = jnp.concatenate([src, padi % N]).reshape(NWORK * NCHUNK, CHUNK)
    dst_p = jnp.concatenate([dst, N + padi % npadrows]).reshape(
        NWORK * NCHUNK, CHUNK)

    parts_deg = _deg_call(dst_p)
    degT = parts_deg[:, :N].T

    hs1, dinv = _tc1(x, W1, degT)
    parts1 = _agg128(hs1, src_p, dst_p)
    hs2 = _tcmid(parts1, hs1, dinv, b1.reshape(1, D), ln1_g.reshape(1, D),
                 ln1_b.reshape(1, D), W2, D)
    parts2 = _agg128(hs2, src_p, dst_p)
    W3p = jnp.pad(W3, ((0, 0), (0, D - DOUT)))
    hs3 = _tcmid(parts2, hs2, dinv, b2.reshape(1, D), ln2_g.reshape(1, D),
                 ln2_b.reshape(1, D), W3p, D)
    parts3 = _agg128(hs3, src_p, dst_p)
    return _tclast(parts3, hs3, dinv, b3.reshape(1, DOUT))
